# Initial kernel scaffold; baseline (speedup 1.0000x reference)
import jax
import jax.numpy as jnp
from jax.experimental import pallas as pl


def kernel(x1, x2, x3, adj1, adj2, adj3, W1, b1, W2, b2, aa_W, na_W, na_b, Wq, bq, Wk, bk, Wv, bv, mha_Win, mha_bin, mha_Wout, mha_bout, fcW, fcb, outW, outb):
    raise NotImplementedError("write your pallas kernel here")


# speedup vs baseline: 18.1801x; 18.1801x over previous
import numpy as np
import jax
import jax.numpy as jnp
from jax.experimental import pallas as pl

B = 16
N = 400
DIN = 400
HID = 256
NN = N * N

_RANK_LOW = 112000
_HW = np.float32(0.296875)
_LW = np.float32(0.703125)
_ONE_BITS = np.int32(0x3F800000)


def _kth_smallest_bits(bits, rank):
    def body(_, lohi):
        lo, hi = lohi
        mid = (lo + hi) >> 1
        c = jnp.sum((bits <= mid).astype(jnp.int32))
        pred = c >= rank
        return (jnp.where(pred, lo, mid + 1), jnp.where(pred, mid, hi))

    lo, hi = jax.lax.fori_loop(
        0, 31, body, (jnp.int32(0), _ONE_BITS), unroll=True)
    return hi


def _gcn_body(x1_ref, x2_ref, x3_ref, a1_ref, a2_ref, a3_ref,
              w1_ref, b1_ref, w2_ref, b2_ref, out_ref):
    xs = (x1_ref, x2_ref, x3_ref)
    adjs = (a1_ref, a2_ref, a3_ref)
    row_ids = jax.lax.broadcasted_iota(jnp.int32, (N, N), 0)
    col_ids = jax.lax.broadcasted_iota(jnp.int32, (N, N), 1)
    eye = (row_ids == col_ids).astype(jnp.float32)
    for i in range(3):
        adj = adjs[i][0]
        x = xs[i][0]
        bits = jax.lax.bitcast_convert_type(adj, jnp.int32)

        a_bits = _kth_smallest_bits(bits, _RANK_LOW)
        c_a = jnp.sum((bits <= a_bits).astype(jnp.int32))
        above = jnp.where(bits > a_bits, bits, _ONE_BITS)
        b_bits = jnp.where(c_a >= _RANK_LOW + 1, a_bits, jnp.min(above))

        ab = jax.lax.bitcast_convert_type(
            jnp.stack([a_bits, b_bits]).reshape(1, 2), jnp.float32)
        thr = ab[0, 0] * _LW + ab[0, 1] * _HW

        A = (adj > thr).astype(jnp.float32)
        Ai = A + eye
        deg = jnp.sum(Ai, axis=1, keepdims=True)
        dinv = jax.lax.rsqrt(deg)
        An = Ai * dinv * dinv.reshape(1, N)

        xw = jnp.dot(x, w1_ref[i], preferred_element_type=jnp.float32)
        h1 = jnp.maximum(
            jnp.dot(An, xw, preferred_element_type=jnp.float32)
            + b1_ref[i:i + 1, :], 0.0)
        hw2 = jnp.dot(h1, w2_ref[i], preferred_element_type=jnp.float32)
        h2 = jnp.maximum(
            jnp.dot(An, hw2, preferred_element_type=jnp.float32)
            + b2_ref[i:i + 1, :], 0.0)
        g = jnp.sum(h2, axis=0, keepdims=True) * np.float32(1.0 / N)
        out_ref[0, :, i * HID:(i + 1) * HID] = g


def _fusion_body(g_ref, aaWT_ref, naW_ref, nab_ref, wvT_ref, bv_ref,
                 winvT_ref, binv_ref, woutT_ref, bout_ref,
                 fcwT_ref, fcb_ref, outWT_ref, outb_ref, out_ref):
    gs = [g_ref[:, 0, i * HID:(i + 1) * HID] for i in range(3)]

    means = jnp.concatenate(
        [jnp.mean(g, axis=1, keepdims=True) for g in gs], axis=1)
    maxes = jnp.concatenate(
        [jnp.max(g, axis=1, keepdims=True) for g in gs], axis=1)
    avg_out = jnp.dot(means, aaWT_ref[...], preferred_element_type=jnp.float32)
    max_out = jnp.dot(maxes, aaWT_ref[...], preferred_element_type=jnp.float32)
    aw = jax.nn.sigmoid(avg_out + max_out)
    wts = [gs[i] * aw[:, i:i + 1] for i in range(3)]
    wtm = (wts[0] + wts[1] + wts[2]) * np.float32(1.0 / 3.0)
    wtx = jnp.maximum(jnp.maximum(wts[0], wts[1]), wts[2])

    zcol = jnp.zeros((B, 1), jnp.float32)
    conv = jnp.broadcast_to(nab_ref[0:1, 0:1], (B, HID))
    for c, arr in enumerate((wtm, wtx)):
        right = jnp.concatenate([zcol, arr[:, :HID - 1]], axis=1)
        left = jnp.concatenate([arr[:, 1:], zcol], axis=1)
        conv = (conv + right * naW_ref[c:c + 1, 0:1]
                + arr * naW_ref[c:c + 1, 1:2]
                + left * naW_ref[c:c + 1, 2:3])
    sig = jax.nn.sigmoid(conv)
    xa = jnp.concatenate([gs[i] + wts[i] * sig for i in range(3)], axis=1)

    fus = []
    for i in range(3):
        v = jnp.dot(gs[i], wvT_ref[i], preferred_element_type=jnp.float32) \
            + bv_ref[i:i + 1, :]
        vv = jnp.dot(v, winvT_ref[...], preferred_element_type=jnp.float32) \
            + binv_ref[...]
        m = jnp.dot(vv, woutT_ref[...], preferred_element_type=jnp.float32) \
            + bout_ref[...]
        eff = fcwT_ref[i, :HID, :] + fcwT_ref[i, HID:, :]
        fus.append(jnp.dot(m, eff, preferred_element_type=jnp.float32)
                   + fcb_ref[i:i + 1, :])

    cat = jnp.concatenate([xa] + fus, axis=1)
    out_ref[...] = jnp.dot(cat, outWT_ref[...],
                           preferred_element_type=jnp.float32) + outb_ref[...]


def kernel(x1, x2, x3, adj1, adj2, adj3, W1, b1, W2, b2, aa_W, na_W, na_b,
           Wq, bq, Wk, bk, Wv, bv, mha_Win, mha_bin, mha_Wout, mha_bout,
           fcW, fcb, outW, outb):
    samp = pl.BlockSpec((1, N, N), lambda b: (b, 0, 0))
    whole = lambda a: pl.BlockSpec(a.shape, lambda *_: (0,) * a.ndim)

    g = pl.pallas_call(
        _gcn_body,
        grid=(B,),
        in_specs=[samp, samp, samp, samp, samp, samp,
                  whole(W1), whole(b1), whole(W2), whole(b2)],
        out_specs=pl.BlockSpec((1, 1, 3 * HID), lambda b: (b, 0, 0)),
        out_shape=jax.ShapeDtypeStruct((B, 1, 3 * HID), jnp.float32),
    )(x1, x2, x3, adj1, adj2, adj3, W1, b1, W2, b2)

    aaWT = aa_W.T
    naW2 = na_W[0]
    nab2 = na_b.reshape(1, 1)
    wvT = jnp.swapaxes(Wv, 1, 2)
    winvT = mha_Win[2 * HID:].T
    binv = mha_bin[2 * HID:].reshape(1, HID)
    woutT = mha_Wout.T
    bout2 = mha_bout.reshape(1, HID)
    fcwT = jnp.swapaxes(fcW, 1, 2)
    outWT = outW.T
    outb2 = outb.reshape(1, 2)

    args = (g, aaWT, naW2, nab2, wvT, bv, winvT, binv, woutT, bout2,
            fcwT, fcb, outWT, outb2)
    out = pl.pallas_call(
        _fusion_body,
        in_specs=[whole(a) for a in args],
        out_specs=pl.BlockSpec((B, 2), lambda: (0, 0)),
        out_shape=jax.ShapeDtypeStruct((B, 2), jnp.float32),
    )(*args)
    return out


# speedup vs baseline: 32.5570x; 1.7908x over previous
import numpy as np
import jax
import jax.numpy as jnp
from jax.experimental import pallas as pl
from jax.experimental.pallas import tpu as pltpu

B = 16
N = 400
DIN = 400
HID = 256
NN = N * N

_RANK_LOW = 112000
_HW = np.float32(0.296875)
_LW = np.float32(0.703125)
_ONE_BITS = np.int32(0x3F800000)


def _gcn_body(x1_ref, x2_ref, x3_ref, a1_ref, a2_ref, a3_ref,
              w1_ref, b1_ref, w2_ref, b2_ref, out_ref):
    xs = (x1_ref, x2_ref, x3_ref)
    adjs_refs = (a1_ref, a2_ref, a3_ref)
    row_ids = jax.lax.broadcasted_iota(jnp.int32, (N, N), 0)
    col_ids = jax.lax.broadcasted_iota(jnp.int32, (N, N), 1)
    eye = (row_ids == col_ids).astype(jnp.float32)

    adjs = [adjs_refs[i][0] for i in range(3)]
    bits3 = [jax.lax.bitcast_convert_type(a, jnp.int32) for a in adjs]

    xws = [jnp.dot(xs[i][0], w1_ref[i], preferred_element_type=jnp.float32)
           for i in range(3)]

    def body(_, carry):
        los, his = carry[:3], carry[3:]
        nlo, nhi = [], []
        for i in range(3):
            mid = (los[i] + his[i]) >> 1
            c = jnp.sum((bits3[i] <= mid).astype(jnp.int32))
            pred = c >= _RANK_LOW
            nlo.append(jnp.where(pred, los[i], mid + 1))
            nhi.append(jnp.where(pred, mid, his[i]))
        return tuple(nlo) + tuple(nhi)

    zero = jnp.int32(0)
    res = jax.lax.fori_loop(
        0, 30, body, (zero, zero, zero, _ONE_BITS, _ONE_BITS, _ONE_BITS),
        unroll=True)
    abits3 = res[3:]

    for i in range(3):
        bits, a_bits = bits3[i], abits3[i]
        c_a = jnp.sum((bits <= a_bits).astype(jnp.int32))
        above = jnp.where(bits > a_bits, bits, _ONE_BITS)
        b_bits = jnp.where(c_a >= _RANK_LOW + 1, a_bits, jnp.min(above))

        ab = jax.lax.bitcast_convert_type(
            jnp.stack([a_bits, b_bits]).reshape(1, 2), jnp.float32)
        thr = ab[0, 0] * _LW + ab[0, 1] * _HW

        A = (adjs[i] > thr).astype(jnp.float32)
        Ai = A + eye
        deg = jnp.sum(Ai, axis=1, keepdims=True)
        dinv = jax.lax.rsqrt(deg)
        An = Ai * dinv * dinv.reshape(1, N)

        h1 = jnp.maximum(
            jnp.dot(An, xws[i], preferred_element_type=jnp.float32)
            + b1_ref[i:i + 1, :], 0.0)
        hw2 = jnp.dot(h1, w2_ref[i], preferred_element_type=jnp.float32)
        h2 = jnp.maximum(
            jnp.dot(An, hw2, preferred_element_type=jnp.float32)
            + b2_ref[i:i + 1, :], 0.0)
        g = jnp.sum(h2, axis=0, keepdims=True) * np.float32(1.0 / N)
        out_ref[0, :, i * HID:(i + 1) * HID] = g


def _fusion_body(g_ref, aaWT_ref, naW_ref, nab_ref, wvT_ref, bv_ref,
                 winvT_ref, binv_ref, woutT_ref, bout_ref,
                 fcwT_ref, fcb_ref, outWT_ref, outb_ref, out_ref):
    gs = [g_ref[:, 0, i * HID:(i + 1) * HID] for i in range(3)]

    means = jnp.concatenate(
        [jnp.mean(g, axis=1, keepdims=True) for g in gs], axis=1)
    maxes = jnp.concatenate(
        [jnp.max(g, axis=1, keepdims=True) for g in gs], axis=1)
    avg_out = jnp.dot(means, aaWT_ref[...], preferred_element_type=jnp.float32)
    max_out = jnp.dot(maxes, aaWT_ref[...], preferred_element_type=jnp.float32)
    aw = jax.nn.sigmoid(avg_out + max_out)
    wts = [gs[i] * aw[:, i:i + 1] for i in range(3)]
    wtm = (wts[0] + wts[1] + wts[2]) * np.float32(1.0 / 3.0)
    wtx = jnp.maximum(jnp.maximum(wts[0], wts[1]), wts[2])

    zcol = jnp.zeros((B, 1), jnp.float32)
    conv = jnp.broadcast_to(nab_ref[0:1, 0:1], (B, HID))
    for c, arr in enumerate((wtm, wtx)):
        right = jnp.concatenate([zcol, arr[:, :HID - 1]], axis=1)
        left = jnp.concatenate([arr[:, 1:], zcol], axis=1)
        conv = (conv + right * naW_ref[c:c + 1, 0:1]
                + arr * naW_ref[c:c + 1, 1:2]
                + left * naW_ref[c:c + 1, 2:3])
    sig = jax.nn.sigmoid(conv)
    xa = jnp.concatenate([gs[i] + wts[i] * sig for i in range(3)], axis=1)

    fus = []
    for i in range(3):
        v = jnp.dot(gs[i], wvT_ref[i], preferred_element_type=jnp.float32) \
            + bv_ref[i:i + 1, :]
        vv = jnp.dot(v, winvT_ref[...], preferred_element_type=jnp.float32) \
            + binv_ref[...]
        m = jnp.dot(vv, woutT_ref[...], preferred_element_type=jnp.float32) \
            + bout_ref[...]
        eff = fcwT_ref[i, :HID, :] + fcwT_ref[i, HID:, :]
        fus.append(jnp.dot(m, eff, preferred_element_type=jnp.float32)
                   + fcb_ref[i:i + 1, :])

    cat = jnp.concatenate([xa] + fus, axis=1)
    out_ref[...] = jnp.dot(cat, outWT_ref[...],
                           preferred_element_type=jnp.float32) + outb_ref[...]


def kernel(x1, x2, x3, adj1, adj2, adj3, W1, b1, W2, b2, aa_W, na_W, na_b,
           Wq, bq, Wk, bk, Wv, bv, mha_Win, mha_bin, mha_Wout, mha_bout,
           fcW, fcb, outW, outb):
    samp = pl.BlockSpec((1, N, N), lambda b: (b, 0, 0))
    whole = lambda a: pl.BlockSpec(a.shape, lambda *_: (0,) * a.ndim)

    g = pl.pallas_call(
        _gcn_body,
        grid=(B,),
        in_specs=[samp, samp, samp, samp, samp, samp,
                  whole(W1), whole(b1), whole(W2), whole(b2)],
        out_specs=pl.BlockSpec((1, 1, 3 * HID), lambda b: (b, 0, 0)),
        out_shape=jax.ShapeDtypeStruct((B, 1, 3 * HID), jnp.float32),
        compiler_params=pltpu.CompilerParams(
            dimension_semantics=("parallel",)),
    )(x1, x2, x3, adj1, adj2, adj3, W1, b1, W2, b2)

    aaWT = aa_W.T
    naW2 = na_W[0]
    nab2 = na_b.reshape(1, 1)
    wvT = jnp.swapaxes(Wv, 1, 2)
    winvT = mha_Win[2 * HID:].T
    binv = mha_bin[2 * HID:].reshape(1, HID)
    woutT = mha_Wout.T
    bout2 = mha_bout.reshape(1, HID)
    fcwT = jnp.swapaxes(fcW, 1, 2)
    outWT = outW.T
    outb2 = outb.reshape(1, 2)

    args = (g, aaWT, naW2, nab2, wvT, bv, winvT, binv, woutT, bout2,
            fcwT, fcb, outWT, outb2)
    out = pl.pallas_call(
        _fusion_body,
        in_specs=[whole(a) for a in args],
        out_specs=pl.BlockSpec((B, 2), lambda: (0, 0)),
        out_shape=jax.ShapeDtypeStruct((B, 2), jnp.float32),
    )(*args)
    return out


# speedup vs baseline: 33.6533x; 1.0337x over previous
import numpy as np
import jax
import jax.numpy as jnp
from jax.experimental import pallas as pl
from jax.experimental.pallas import tpu as pltpu

B = 16
N = 400
DIN = 400
HID = 256
NN = N * N

_RANK_LOW = 112000
_HW = np.float32(0.296875)
_LW = np.float32(0.703125)
_ONE_BITS = np.int32(0x3F800000)


def _dot_t(a, b):
    return jax.lax.dot_general(a, b, (((1,), (1,)), ((), ())),
                               preferred_element_type=jnp.float32)


def _fusion(g, aa_ref, naW_ref, nab_ref, wv_ref, bv_ref, win_ref, bin_ref,
            wout_ref, bout_ref, fcw_ref, fcb_ref, outw_ref, outb_ref):
    gs = [g[:, i * HID:(i + 1) * HID] for i in range(3)]

    means = jnp.concatenate(
        [jnp.mean(x, axis=1, keepdims=True) for x in gs], axis=1)
    maxes = jnp.concatenate(
        [jnp.max(x, axis=1, keepdims=True) for x in gs], axis=1)
    aw = jax.nn.sigmoid(_dot_t(means, aa_ref[...])
                        + _dot_t(maxes, aa_ref[...]))
    wts = [gs[i] * aw[:, i:i + 1] for i in range(3)]
    wtm = (wts[0] + wts[1] + wts[2]) * np.float32(1.0 / 3.0)
    wtx = jnp.maximum(jnp.maximum(wts[0], wts[1]), wts[2])

    zcol = jnp.zeros((B, 1), jnp.float32)
    conv = jnp.broadcast_to(nab_ref[0:1, 0:1], (B, HID))
    for c, arr in enumerate((wtm, wtx)):
        right = jnp.concatenate([zcol, arr[:, :HID - 1]], axis=1)
        left = jnp.concatenate([arr[:, 1:], zcol], axis=1)
        conv = (conv + right * naW_ref[c:c + 1, 0:1]
                + arr * naW_ref[c:c + 1, 1:2]
                + left * naW_ref[c:c + 1, 2:3])
    sig = jax.nn.sigmoid(conv)
    xa = jnp.concatenate([gs[i] + wts[i] * sig for i in range(3)], axis=1)

    fus = []
    for i in range(3):
        v = _dot_t(gs[i], wv_ref[i]) + bv_ref[i:i + 1, :]
        vv = _dot_t(v, win_ref[2 * HID:, :]) + bin_ref[:, 2 * HID:]
        m = _dot_t(vv, wout_ref[...]) + bout_ref[...]
        fus.append(_dot_t(m, fcw_ref[i, :, :HID] + fcw_ref[i, :, HID:])
                   + fcb_ref[i:i + 1, :])

    cat = jnp.concatenate([xa] + fus, axis=1)
    return _dot_t(cat, outw_ref[...]) + outb_ref[...]


def _body(x1_ref, x2_ref, x3_ref, a1_ref, a2_ref, a3_ref,
          w1_ref, b1_ref, w2_ref, b2_ref,
          aa_ref, naW_ref, nab_ref, wv_ref, bv_ref, win_ref, bin_ref,
          wout_ref, bout_ref, fcw_ref, fcb_ref, outw_ref, outb_ref,
          out_ref, g_scratch):
    xs = (x1_ref, x2_ref, x3_ref)
    row_ids = jax.lax.broadcasted_iota(jnp.int32, (N, N), 0)
    col_ids = jax.lax.broadcasted_iota(jnp.int32, (N, N), 1)
    eye = (row_ids == col_ids).astype(jnp.float32)

    adjs = [r[0] for r in (a1_ref, a2_ref, a3_ref)]
    bits3 = [jax.lax.bitcast_convert_type(a, jnp.int32) for a in adjs]

    xws = [jnp.dot(xs[i][0], w1_ref[i], preferred_element_type=jnp.float32)
           for i in range(3)]

    def body(_, carry):
        los, his = carry[:3], carry[3:]
        nlo, nhi = [], []
        for i in range(3):
            mid = (los[i] + his[i]) >> 1
            c = jnp.sum((bits3[i] <= mid).astype(jnp.int32))
            pred = c >= _RANK_LOW
            nlo.append(jnp.where(pred, los[i], mid + 1))
            nhi.append(jnp.where(pred, mid, his[i]))
        return tuple(nlo) + tuple(nhi)

    zero = jnp.int32(0)
    res = jax.lax.fori_loop(
        0, 30, body, (zero, zero, zero, _ONE_BITS, _ONE_BITS, _ONE_BITS),
        unroll=True)
    abits3 = res[3:]

    b_idx = pl.program_id(0)
    for i in range(3):
        bits, a_bits = bits3[i], abits3[i]
        c_a = jnp.sum((bits <= a_bits).astype(jnp.int32))
        above = jnp.where(bits > a_bits, bits, _ONE_BITS)
        b_bits = jnp.where(c_a >= _RANK_LOW + 1, a_bits, jnp.min(above))

        ab = jax.lax.bitcast_convert_type(
            jnp.stack([a_bits, b_bits]).reshape(1, 2), jnp.float32)
        thr = ab[0, 0] * _LW + ab[0, 1] * _HW

        A = (adjs[i] > thr).astype(jnp.float32)
        Ai = A + eye
        deg = jnp.sum(Ai, axis=1, keepdims=True)
        dinv = jax.lax.rsqrt(deg)
        An = Ai * dinv * dinv.reshape(1, N)

        h1 = jnp.maximum(
            jnp.dot(An, xws[i], preferred_element_type=jnp.float32)
            + b1_ref[i:i + 1, :], 0.0)
        hw2 = jnp.dot(h1, w2_ref[i], preferred_element_type=jnp.float32)
        h2 = jnp.maximum(
            jnp.dot(An, hw2, preferred_element_type=jnp.float32)
            + b2_ref[i:i + 1, :], 0.0)
        g = jnp.sum(h2, axis=0, keepdims=True) * np.float32(1.0 / N)
        g_scratch[pl.ds(b_idx, 1), i * HID:(i + 1) * HID] = g

    @pl.when(b_idx == B - 1)
    def _():
        out_ref[...] = _fusion(
            g_scratch[...], aa_ref, naW_ref, nab_ref, wv_ref, bv_ref,
            win_ref, bin_ref, wout_ref, bout_ref, fcw_ref, fcb_ref,
            outw_ref, outb_ref)


def kernel(x1, x2, x3, adj1, adj2, adj3, W1, b1, W2, b2, aa_W, na_W, na_b,
           Wq, bq, Wk, bk, Wv, bv, mha_Win, mha_bin, mha_Wout, mha_bout,
           fcW, fcb, outW, outb):
    samp = pl.BlockSpec((1, N, N), lambda b: (b, 0, 0))
    whole = lambda a: pl.BlockSpec(a.shape, lambda b: (0,) * a.ndim)

    naW2 = na_W[0]
    nab2 = na_b.reshape(1, 1)
    bin2 = mha_bin.reshape(1, 3 * HID)
    bout2 = mha_bout.reshape(1, HID)
    outb2 = outb.reshape(1, 2)

    weights = (W1, b1, W2, b2, aa_W, naW2, nab2, Wv, bv, mha_Win, bin2,
               mha_Wout, bout2, fcW, fcb, outW, outb2)
    out = pl.pallas_call(
        _body,
        grid=(B,),
        in_specs=[samp] * 6 + [whole(w) for w in weights],
        out_specs=pl.BlockSpec((B, 2), lambda b: (0, 0)),
        out_shape=jax.ShapeDtypeStruct((B, 2), jnp.float32),
        scratch_shapes=[pltpu.VMEM((B, 3 * HID), jnp.float32)],
    )(x1, x2, x3, adj1, adj2, adj3, *weights)
    return out


# speedup vs baseline: 56.5576x; 1.6806x over previous
import numpy as np
import jax
import jax.numpy as jnp
from jax.experimental import pallas as pl
from jax.experimental.pallas import tpu as pltpu

B = 16
SPB = 2
N = 400
DIN = 400
HID = 256
NN = N * N

_RANK_LOW = 112000
_HW = np.float32(0.296875)
_LW = np.float32(0.703125)
_ONE_BITS = np.int32(0x3F800000)


def _dot_t(a, b):
    return jax.lax.dot_general(a, b, (((1,), (1,)), ((), ())),
                               preferred_element_type=jnp.float32)


def _count16(data, pred_fn):
    parts = [pred_fn(data[16 * j:16 * (j + 1)]).astype(jnp.int16)
             for j in range(25)]
    while len(parts) > 1:
        nxt = [parts[k] + parts[k + 1] for k in range(0, len(parts) - 1, 2)]
        if len(parts) % 2:
            nxt.append(parts[-1])
        parts = nxt
    return jnp.sum(parts[0].astype(jnp.int32))


def _fusion(g, aa_ref, naW_ref, nab_ref, wv_ref, bv_ref, win_ref, bin_ref,
            wout_ref, bout_ref, fcw_ref, fcb_ref, outw_ref, outb_ref):
    gs = [g[:, i * HID:(i + 1) * HID] for i in range(3)]

    means = jnp.concatenate(
        [jnp.mean(x, axis=1, keepdims=True) for x in gs], axis=1)
    maxes = jnp.concatenate(
        [jnp.max(x, axis=1, keepdims=True) for x in gs], axis=1)
    aw = jax.nn.sigmoid(_dot_t(means, aa_ref[...])
                        + _dot_t(maxes, aa_ref[...]))
    wts = [gs[i] * aw[:, i:i + 1] for i in range(3)]
    wtm = (wts[0] + wts[1] + wts[2]) * np.float32(1.0 / 3.0)
    wtx = jnp.maximum(jnp.maximum(wts[0], wts[1]), wts[2])

    zcol = jnp.zeros((B, 1), jnp.float32)
    conv = jnp.broadcast_to(nab_ref[0:1, 0:1], (B, HID))
    for c, arr in enumerate((wtm, wtx)):
        right = jnp.concatenate([zcol, arr[:, :HID - 1]], axis=1)
        left = jnp.concatenate([arr[:, 1:], zcol], axis=1)
        conv = (conv + right * naW_ref[c:c + 1, 0:1]
                + arr * naW_ref[c:c + 1, 1:2]
                + left * naW_ref[c:c + 1, 2:3])
    sig = jax.nn.sigmoid(conv)
    xa = jnp.concatenate([gs[i] + wts[i] * sig for i in range(3)], axis=1)

    fus = []
    for i in range(3):
        v = _dot_t(gs[i], wv_ref[i]) + bv_ref[i:i + 1, :]
        vv = _dot_t(v, win_ref[2 * HID:, :]) + bin_ref[:, 2 * HID:]
        m = _dot_t(vv, wout_ref[...]) + bout_ref[...]
        fus.append(_dot_t(m, fcw_ref[i, :, :HID] + fcw_ref[i, :, HID:])
                   + fcb_ref[i:i + 1, :])

    cat = jnp.concatenate([xa] + fus, axis=1)
    return _dot_t(cat, outw_ref[...]) + outb_ref[...]


def _body(x1_ref, x2_ref, x3_ref, a1_ref, a2_ref, a3_ref,
          w1_ref, b1_ref, w2_ref, b2_ref,
          aa_ref, naW_ref, nab_ref, wv_ref, bv_ref, win_ref, bin_ref,
          wout_ref, bout_ref, fcw_ref, fcb_ref, outw_ref, outb_ref,
          out_ref, g_scratch):
    xs = (x1_ref, x2_ref, x3_ref)
    row_ids = jax.lax.broadcasted_iota(jnp.int32, (N, N), 0)
    col_ids = jax.lax.broadcasted_iota(jnp.int32, (N, N), 1)
    eye = (row_ids == col_ids).astype(jnp.float32)

    adjs, bits_u, xws = [], [], []
    for s in range(SPB):
        for i in range(3):
            a = (a1_ref, a2_ref, a3_ref)[i][s]
            adjs.append(a)
            bits_u.append(jax.lax.bitcast_convert_type(a, jnp.int32))
            xws.append(jnp.dot(xs[i][s], w1_ref[i],
                               preferred_element_type=jnp.float32))
    U = SPB * 3

    hi16 = [(b >> 16).astype(jnp.int16) for b in bits_u]
    lo16 = [((b & 0xFFFF) - 32768).astype(jnp.int16) for b in bits_u]

    def phase1(_, carry):
        los, his = carry[:U], carry[U:]
        nlo, nhi = [], []
        for u in range(U):
            mid = (los[u] + his[u]) >> 1
            m16 = mid.astype(jnp.int16)
            c = _count16(hi16[u], lambda d: d <= m16)
            pred = c >= _RANK_LOW
            nlo.append(jnp.where(pred, los[u], mid + 1))
            nhi.append(jnp.where(pred, mid, his[u]))
        return tuple(nlo) + tuple(nhi)

    zero = jnp.int32(0)
    top = jnp.int32(_ONE_BITS >> 16)
    res = jax.lax.fori_loop(0, 14, phase1,
                            (zero,) * U + (top,) * U, unroll=True)
    pU = res[U:]

    rank2, mlo = [], []
    for u in range(U):
        pm1 = (pU[u] - 1).astype(jnp.int16)
        c_below = _count16(hi16[u], lambda d: d <= pm1)
        rank2.append(_RANK_LOW - c_below)
        mlo.append(jnp.where(hi16[u] == pU[u].astype(jnp.int16),
                             lo16[u], jnp.int16(32767)))

    def phase2(_, carry):
        los, his = carry[:U], carry[U:]
        nlo, nhi = [], []
        for u in range(U):
            mid = (los[u] + his[u]) >> 1
            m16 = mid.astype(jnp.int16)
            c = _count16(mlo[u], lambda d: d <= m16)
            pred = c >= rank2[u]
            nlo.append(jnp.where(pred, los[u], mid + 1))
            nhi.append(jnp.where(pred, mid, his[u]))
        return tuple(nlo) + tuple(nhi)

    neg, pos = jnp.int32(-32768), jnp.int32(32767)
    res2 = jax.lax.fori_loop(0, 16, phase2,
                             (neg,) * U + (pos,) * U, unroll=True)
    abits_u = [(pU[u] << 16) + (res2[U + u] + 32768) for u in range(U)]

    step = pl.program_id(0)
    for u in range(U):
        s, i = divmod(u, 3)
        bits, a_bits = bits_u[u], abits_u[u]
        c_a = jnp.sum((bits <= a_bits).astype(jnp.int32))
        above = jnp.where(bits > a_bits, bits, _ONE_BITS)
        b_bits = jnp.where(c_a >= _RANK_LOW + 1, a_bits, jnp.min(above))

        ab = jax.lax.bitcast_convert_type(
            jnp.stack([a_bits, b_bits]).reshape(1, 2), jnp.float32)
        thr = ab[0, 0] * _LW + ab[0, 1] * _HW

        A = (adjs[u] > thr).astype(jnp.float32)
        Ai = A + eye
        deg = jnp.sum(Ai, axis=1, keepdims=True)
        dinv = jax.lax.rsqrt(deg)
        An = Ai * dinv * dinv.reshape(1, N)

        h1 = jnp.maximum(
            jnp.dot(An, xws[u], preferred_element_type=jnp.float32)
            + b1_ref[i:i + 1, :], 0.0)
        hw2 = jnp.dot(h1, w2_ref[i], preferred_element_type=jnp.float32)
        h2 = jnp.maximum(
            jnp.dot(An, hw2, preferred_element_type=jnp.float32)
            + b2_ref[i:i + 1, :], 0.0)
        g = jnp.sum(h2, axis=0, keepdims=True) * np.float32(1.0 / N)
        g_scratch[pl.ds(step * SPB + s, 1), i * HID:(i + 1) * HID] = g

    @pl.when(step == B // SPB - 1)
    def _():
        out_ref[...] = _fusion(
            g_scratch[...], aa_ref, naW_ref, nab_ref, wv_ref, bv_ref,
            win_ref, bin_ref, wout_ref, bout_ref, fcw_ref, fcb_ref,
            outw_ref, outb_ref)


def kernel(x1, x2, x3, adj1, adj2, adj3, W1, b1, W2, b2, aa_W, na_W, na_b,
           Wq, bq, Wk, bk, Wv, bv, mha_Win, mha_bin, mha_Wout, mha_bout,
           fcW, fcb, outW, outb):
    samp = pl.BlockSpec((SPB, N, N), lambda b: (b, 0, 0))
    whole = lambda a: pl.BlockSpec(a.shape, lambda b: (0,) * a.ndim)

    naW2 = na_W[0]
    nab2 = na_b.reshape(1, 1)
    bin2 = mha_bin.reshape(1, 3 * HID)
    bout2 = mha_bout.reshape(1, HID)
    outb2 = outb.reshape(1, 2)

    weights = (W1, b1, W2, b2, aa_W, naW2, nab2, Wv, bv, mha_Win, bin2,
               mha_Wout, bout2, fcW, fcb, outW, outb2)
    out = pl.pallas_call(
        _body,
        grid=(B // SPB,),
        in_specs=[samp] * 6 + [whole(w) for w in weights],
        out_specs=pl.BlockSpec((B, 2), lambda b: (0, 0)),
        out_shape=jax.ShapeDtypeStruct((B, 2), jnp.float32),
        scratch_shapes=[pltpu.VMEM((B, 3 * HID), jnp.float32)],
    )(x1, x2, x3, adj1, adj2, adj3, *weights)
    return out


# speedup vs baseline: 60.7016x; 1.0733x over previous
import numpy as np
import jax
import jax.numpy as jnp
from jax.experimental import pallas as pl
from jax.experimental.pallas import tpu as pltpu

B = 16
SPB = 4
N = 400
DIN = 400
HID = 256
NN = N * N

_RANK_LOW = 112000
_HW = np.float32(0.296875)
_LW = np.float32(0.703125)
_ONE_BITS = np.int32(0x3F800000)


def _dot_t(a, b):
    return jax.lax.dot_general(a, b, (((1,), (1,)), ((), ())),
                               preferred_element_type=jnp.float32)


def _count16(data, pred_fn):
    parts = [pred_fn(data[16 * j:16 * (j + 1)]).astype(jnp.int16)
             for j in range(25)]
    while len(parts) > 1:
        nxt = [parts[k] + parts[k + 1] for k in range(0, len(parts) - 1, 2)]
        if len(parts) % 2:
            nxt.append(parts[-1])
        parts = nxt
    return jnp.sum(parts[0].astype(jnp.int32))


def _fusion(g, aa_ref, naW_ref, nab_ref, wv_ref, bv_ref, win_ref, bin_ref,
            wout_ref, bout_ref, fcw_ref, fcb_ref, outw_ref, outb_ref):
    gs = [g[:, i * HID:(i + 1) * HID] for i in range(3)]

    means = jnp.concatenate(
        [jnp.mean(x, axis=1, keepdims=True) for x in gs], axis=1)
    maxes = jnp.concatenate(
        [jnp.max(x, axis=1, keepdims=True) for x in gs], axis=1)
    aw = jax.nn.sigmoid(_dot_t(means, aa_ref[...])
                        + _dot_t(maxes, aa_ref[...]))
    wts = [gs[i] * aw[:, i:i + 1] for i in range(3)]
    wtm = (wts[0] + wts[1] + wts[2]) * np.float32(1.0 / 3.0)
    wtx = jnp.maximum(jnp.maximum(wts[0], wts[1]), wts[2])

    zcol = jnp.zeros((B, 1), jnp.float32)
    conv = jnp.broadcast_to(nab_ref[0:1, 0:1], (B, HID))
    for c, arr in enumerate((wtm, wtx)):
        right = jnp.concatenate([zcol, arr[:, :HID - 1]], axis=1)
        left = jnp.concatenate([arr[:, 1:], zcol], axis=1)
        conv = (conv + right * naW_ref[c:c + 1, 0:1]
                + arr * naW_ref[c:c + 1, 1:2]
                + left * naW_ref[c:c + 1, 2:3])
    sig = jax.nn.sigmoid(conv)
    xa = jnp.concatenate([gs[i] + wts[i] * sig for i in range(3)], axis=1)

    fus = []
    for i in range(3):
        v = _dot_t(gs[i], wv_ref[i]) + bv_ref[i:i + 1, :]
        vv = _dot_t(v, win_ref[2 * HID:, :]) + bin_ref[:, 2 * HID:]
        m = _dot_t(vv, wout_ref[...]) + bout_ref[...]
        fus.append(_dot_t(m, fcw_ref[i, :, :HID] + fcw_ref[i, :, HID:])
                   + fcb_ref[i:i + 1, :])

    cat = jnp.concatenate([xa] + fus, axis=1)
    return _dot_t(cat, outw_ref[...]) + outb_ref[...]


def _body(x1_ref, x2_ref, x3_ref, a1_ref, a2_ref, a3_ref,
          w1_ref, b1_ref, w2_ref, b2_ref,
          aa_ref, naW_ref, nab_ref, wv_ref, bv_ref, win_ref, bin_ref,
          wout_ref, bout_ref, fcw_ref, fcb_ref, outw_ref, outb_ref,
          out_ref, g_scratch):
    xs = (x1_ref, x2_ref, x3_ref)
    row_ids = jax.lax.broadcasted_iota(jnp.int32, (N, N), 0)
    col_ids = jax.lax.broadcasted_iota(jnp.int32, (N, N), 1)
    eye = (row_ids == col_ids).astype(jnp.float32)

    adjs, hi16, lo16, xws = [], [], [], []
    for s in range(SPB):
        for i in range(3):
            a = (a1_ref, a2_ref, a3_ref)[i][s]
            adjs.append(a)
            b = jax.lax.bitcast_convert_type(a, jnp.int32)
            hi16.append((b >> 16).astype(jnp.int16))
            lo16.append(((b & 0xFFFF) - 32768).astype(jnp.int16))
            xws.append(jnp.dot(xs[i][s], w1_ref[i],
                               preferred_element_type=jnp.float32))
    U = SPB * 3

    def phase1(_, carry):
        los, his = carry[:U], carry[U:]
        nlo, nhi = [], []
        for u in range(U):
            mid = (los[u] + his[u]) >> 1
            m16 = mid.astype(jnp.int16)
            c = _count16(hi16[u], lambda d: d <= m16)
            pred = c >= _RANK_LOW
            nlo.append(jnp.where(pred, los[u], mid + 1))
            nhi.append(jnp.where(pred, mid, his[u]))
        return tuple(nlo) + tuple(nhi)

    zero = jnp.int32(0)
    top = jnp.int32(_ONE_BITS >> 16)
    res = jax.lax.fori_loop(0, 14, phase1,
                            (zero,) * U + (top,) * U, unroll=True)
    pU = res[U:]

    rank2, mlo = [], []
    for u in range(U):
        pm1 = (pU[u] - 1).astype(jnp.int16)
        c_below = _count16(hi16[u], lambda d: d <= pm1)
        rank2.append(_RANK_LOW - c_below)
        mlo.append(jnp.where(hi16[u] == pU[u].astype(jnp.int16),
                             lo16[u], jnp.int16(32767)))

    def phase2(_, carry):
        los, his = carry[:U], carry[U:]
        nlo, nhi = [], []
        for u in range(U):
            mid = (los[u] + his[u]) >> 1
            m16 = mid.astype(jnp.int16)
            c = _count16(mlo[u], lambda d: d <= m16)
            pred = c >= rank2[u]
            nlo.append(jnp.where(pred, los[u], mid + 1))
            nhi.append(jnp.where(pred, mid, his[u]))
        return tuple(nlo) + tuple(nhi)

    neg, pos = jnp.int32(-32768), jnp.int32(32767)
    res2 = jax.lax.fori_loop(0, 16, phase2,
                             (neg,) * U + (pos,) * U, unroll=True)
    abits_u = [(pU[u] << 16) + (res2[U + u] + 32768) for u in range(U)]

    step = pl.program_id(0)
    for u in range(U):
        s, i = divmod(u, 3)
        a_bits = abits_u[u]
        av = jax.lax.bitcast_convert_type(
            jnp.stack([a_bits, a_bits]).reshape(1, 2), jnp.float32)[0, 0]
        m16 = res2[U + u].astype(jnp.int16)
        c_a = (_RANK_LOW - rank2[u]) + _count16(mlo[u], lambda d: d <= m16)
        above = jnp.where(adjs[u] > av, adjs[u], np.float32(1.0))
        bv_ = jnp.where(c_a >= _RANK_LOW + 1, av, jnp.min(above))

        thr = av * _LW + bv_ * _HW

        A = (adjs[u] > thr).astype(jnp.float32)
        Ai = A + eye
        deg = jnp.sum(Ai, axis=1, keepdims=True)
        dinv = jax.lax.rsqrt(deg)
        An = Ai * dinv * dinv.reshape(1, N)

        h1 = jnp.maximum(
            jnp.dot(An, xws[u], preferred_element_type=jnp.float32)
            + b1_ref[i:i + 1, :], 0.0)
        hw2 = jnp.dot(h1, w2_ref[i], preferred_element_type=jnp.float32)
        h2 = jnp.maximum(
            jnp.dot(An, hw2, preferred_element_type=jnp.float32)
            + b2_ref[i:i + 1, :], 0.0)
        g = jnp.sum(h2, axis=0, keepdims=True) * np.float32(1.0 / N)
        g_scratch[pl.ds(step * SPB + s, 1), i * HID:(i + 1) * HID] = g

    @pl.when(step == B // SPB - 1)
    def _():
        out_ref[...] = _fusion(
            g_scratch[...], aa_ref, naW_ref, nab_ref, wv_ref, bv_ref,
            win_ref, bin_ref, wout_ref, bout_ref, fcw_ref, fcb_ref,
            outw_ref, outb_ref)


def kernel(x1, x2, x3, adj1, adj2, adj3, W1, b1, W2, b2, aa_W, na_W, na_b,
           Wq, bq, Wk, bk, Wv, bv, mha_Win, mha_bin, mha_Wout, mha_bout,
           fcW, fcb, outW, outb):
    samp = pl.BlockSpec((SPB, N, N), lambda b: (b, 0, 0))
    whole = lambda a: pl.BlockSpec(a.shape, lambda b: (0,) * a.ndim)

    naW2 = na_W[0]
    nab2 = na_b.reshape(1, 1)
    bin2 = mha_bin.reshape(1, 3 * HID)
    bout2 = mha_bout.reshape(1, HID)
    outb2 = outb.reshape(1, 2)

    weights = (W1, b1, W2, b2, aa_W, naW2, nab2, Wv, bv, mha_Win, bin2,
               mha_Wout, bout2, fcW, fcb, outW, outb2)
    out = pl.pallas_call(
        _body,
        grid=(B // SPB,),
        in_specs=[samp] * 6 + [whole(w) for w in weights],
        out_specs=pl.BlockSpec((B, 2), lambda b: (0, 0)),
        out_shape=jax.ShapeDtypeStruct((B, 2), jnp.float32),
        scratch_shapes=[pltpu.VMEM((B, 3 * HID), jnp.float32)],
    )(x1, x2, x3, adj1, adj2, adj3, *weights)
    return out
